# baseline (speedup 1.0000x reference)
import functools

import jax
import jax.numpy as jnp
from jax.experimental import pallas as pl
from jax.experimental.pallas import tpu as pltpu

B, N, A = 4, 2048, 4
DIM_NODES, DIM_EDGES, K, NUM_RBF = 256, 128, 30, 32
TILE = 128
SIGMA = 20.0 / NUM_RBF


def _body(x12_ref, xt_ref, mrow_ref, mcol_ref, wn_c_ref, wn_l_ref,
          we_rbf_ref, we_dir_ref, bn_ref, be_ref, cen_ref,
          nh_ref, eh_ref, ei_ref, mij_ref, dm_s, g_s):
    i = pl.program_id(1)
    x12 = x12_ref[0]
    xt = xt_ref[0]

    xi = [(((x12[:, c:c + 1] + x12[:, 3 + c:4 + c])
            + x12[:, 6 + c:7 + c]) + x12[:, 9 + c:10 + c]) * 0.25
          for c in range(3)]
    xj = [(((xt[c:c + 1, :] + xt[3 + c:4 + c, :])
            + xt[6 + c:7 + c, :]) + xt[9 + c:10 + c, :]) * 0.25
          for c in range(3)]

    d2 = ((xi[0] - xj[0]) ** 2 + (xi[2] - xj[2]) ** 2) + (xi[1] - xj[1]) ** 2
    dist = jnp.sqrt(d2 + 1e-8)

    col = jax.lax.broadcasted_iota(jnp.int32, (TILE, N), 1)
    row = jax.lax.broadcasted_iota(jnp.int32, (TILE, 1), 0) + i * TILE
    dm_s[...] = jnp.where(col == row, 1e9, dist)

    vt = jnp.concatenate([xj[0], xj[1], xj[2], mrow_ref[0]], axis=0)
    hi = vt.astype(jnp.bfloat16)
    rem = vt - hi.astype(jnp.float32)
    mid = rem.astype(jnp.bfloat16)
    lo = (rem - mid.astype(jnp.float32)).astype(jnp.bfloat16)
    col_row = jax.lax.broadcasted_iota(jnp.int32, (1, N), 1)
    col_hi = (col_row >> 8).astype(jnp.bfloat16)
    col_lo = (col_row & 255).astype(jnp.bfloat16)
    ones_row = jnp.ones((1, N), jnp.bfloat16)
    vt3 = jnp.concatenate([hi, mid, lo, col_hi, col_lo, ones_row,
                           jnp.zeros((1, N), jnp.bfloat16)], axis=0)

    mi = mcol_ref[0]
    cen = cen_ref[0:1, :]
    be = be_ref[0:1, :]
    we_rbf = we_rbf_ref[...]
    we_dir = we_dir_ref[...]

    for k in range(K):
        dmv = dm_s[...]
        m = jnp.min(dmv, axis=1, keepdims=True)
        is_min = dmv == m
        g_s[...] = jax.lax.dot_general(
            is_min.astype(jnp.bfloat16), vt3, (((1,), (1,)), ((), ())),
            preferred_element_type=jnp.float32)
        dm_s[...] = jnp.where(is_min, 1e9, dmv)
        cnt = g_s[:, 14:15]

        @pl.when(jnp.max(cnt) > 1.5)
        def _slow():
            idx2 = jnp.min(jnp.where(is_min, col, N), axis=1,
                           keepdims=True)
            oh = col == idx2
            g_s[...] = jax.lax.dot_general(
                oh.astype(jnp.bfloat16), vt3, (((1,), (1,)), ((), ())),
                preferred_element_type=jnp.float32)
            dm_s[...] = jnp.where(oh, 1e9, dmv)

        g = g_s[...]
        idx = (g[:, 12:13] * 256.0 + g[:, 13:14]).astype(jnp.int32)
        dx = ((g[:, 0:1] + g[:, 4:5]) + g[:, 8:9]) - xi[0]
        dy = ((g[:, 1:2] + g[:, 5:6]) + g[:, 9:10]) - xi[1]
        dz = ((g[:, 2:3] + g[:, 6:7]) + g[:, 10:11]) - xi[2]
        mj = g[:, 3:4]
        inv = 1.0 / (m + 1e-8)
        z = (m - cen) / SIGMA
        rbf = jnp.exp(-(z * z))
        eh = jax.lax.dot_general(
            rbf, we_rbf, (((1,), (0,)), ((), ())),
            preferred_element_type=jnp.float32)
        eh = eh + ((dx * inv) * we_dir[0:1, :]
                   + (dy * inv) * we_dir[1:2, :]
                   + (dz * inv) * we_dir[2:3, :]) + be
        mij = mi * mj
        eh_ref[0, :, k * DIM_EDGES:(k + 1) * DIM_EDGES] = eh * mij
        ei_ref[0, :, k:k + 1] = idx
        mij_ref[0, :, k:k + 1] = mij

    xc3 = jnp.concatenate([xi[0], xi[1], xi[2]], axis=1)
    cen4 = jnp.concatenate([xc3, xc3, xc3, xc3], axis=1)
    centered = x12 - cen4
    lens = []
    for a in range(A):
        ca = centered[:, a * 3:a * 3 + 3]
        lens.append(jnp.sqrt(jnp.sum(ca * ca, axis=1, keepdims=True)))
    logl = jnp.log(jnp.concatenate(lens, axis=1) + 1e-6)
    nh = jax.lax.dot_general(
        centered, wn_c_ref[...], (((1,), (0,)), ((), ())),
        preferred_element_type=jnp.float32)
    nh = nh + jax.lax.dot_general(
        logl, wn_l_ref[...], (((1,), (0,)), ((), ())),
        preferred_element_type=jnp.float32)
    nh_ref[0] = (nh + bn_ref[0:1, :]) * mi


@functools.partial(jax.jit, static_argnames=())
def kernel(X, C, W_node, b_node, W_edge, b_edge):
    x12 = X.reshape(B, N, A * 3)
    xt = x12.transpose(0, 2, 1)
    mask_i = (C >= 0).astype(jnp.float32)
    mrow = mask_i.reshape(B, 1, N)
    mcol = mask_i.reshape(B, N, 1)
    centers = jnp.linspace(0.0, 20.0, NUM_RBF).astype(jnp.float32).reshape(1, NUM_RBF)

    grid = (B, N // TILE)
    node_h, edge_big, edge_idx, mask_ij = pl.pallas_call(
        _body,
        grid=grid,
        in_specs=[
            pl.BlockSpec((1, TILE, A * 3), lambda b, i: (b, i, 0)),
            pl.BlockSpec((1, A * 3, N), lambda b, i: (b, 0, 0)),
            pl.BlockSpec((1, 1, N), lambda b, i: (b, 0, 0)),
            pl.BlockSpec((1, TILE, 1), lambda b, i: (b, i, 0)),
            pl.BlockSpec((A * 3, DIM_NODES), lambda b, i: (0, 0)),
            pl.BlockSpec((A, DIM_NODES), lambda b, i: (0, 0)),
            pl.BlockSpec((NUM_RBF, DIM_EDGES), lambda b, i: (0, 0)),
            pl.BlockSpec((3, DIM_EDGES), lambda b, i: (0, 0)),
            pl.BlockSpec((1, DIM_NODES), lambda b, i: (0, 0)),
            pl.BlockSpec((1, DIM_EDGES), lambda b, i: (0, 0)),
            pl.BlockSpec((1, NUM_RBF), lambda b, i: (0, 0)),
        ],
        out_specs=[
            pl.BlockSpec((1, TILE, DIM_NODES), lambda b, i: (b, i, 0)),
            pl.BlockSpec((1, TILE, K * DIM_EDGES), lambda b, i: (b, i, 0)),
            pl.BlockSpec((1, TILE, K), lambda b, i: (b, i, 0)),
            pl.BlockSpec((1, TILE, K), lambda b, i: (b, i, 0)),
        ],
        out_shape=[
            jax.ShapeDtypeStruct((B, N, DIM_NODES), jnp.float32),
            jax.ShapeDtypeStruct((B, N, K * DIM_EDGES), jnp.float32),
            jax.ShapeDtypeStruct((B, N, K), jnp.int32),
            jax.ShapeDtypeStruct((B, N, K), jnp.float32),
        ],
        scratch_shapes=[
            pltpu.VMEM((TILE, N), jnp.float32),
            pltpu.VMEM((TILE, 16), jnp.float32),
        ],
    )(x12, xt, mrow, mcol,
      W_node[:A * 3], W_node[A * 3:],
      W_edge[:NUM_RBF], W_edge[NUM_RBF:],
      b_node.reshape(1, DIM_NODES), b_edge.reshape(1, DIM_EDGES), centers)

    edge_h = edge_big.reshape(B, N, K, DIM_EDGES)
    return node_h, edge_h, edge_idx, mask_i, mask_ij


# speedup vs baseline: 1.1656x; 1.1656x over previous
import functools

import jax
import jax.numpy as jnp
from jax.experimental import pallas as pl
from jax.experimental.pallas import tpu as pltpu

B, N, A = 4, 2048, 4
DIM_NODES, DIM_EDGES, K, NUM_RBF = 256, 128, 30, 32
TILE = 128
SIGMA = 20.0 / NUM_RBF


def _body(x12_ref, xt_ref, mrow_ref, mcol_ref, wn_c_ref, wn_l_ref,
          we_rbf_ref, we_dir_ref, bn_ref, be_ref, cen_ref,
          nh_ref, eh_ref, ei_ref, mij_ref):
    i = pl.program_id(1)
    x12 = x12_ref[0]
    xt = xt_ref[0]

    xi = [(((x12[:, c:c + 1] + x12[:, 3 + c:4 + c])
            + x12[:, 6 + c:7 + c]) + x12[:, 9 + c:10 + c]) * 0.25
          for c in range(3)]
    xj = [(((xt[c:c + 1, :] + xt[3 + c:4 + c, :])
            + xt[6 + c:7 + c, :]) + xt[9 + c:10 + c, :]) * 0.25
          for c in range(3)]

    d2 = ((xi[0] - xj[0]) ** 2 + (xi[2] - xj[2]) ** 2) + (xi[1] - xj[1]) ** 2
    dist = jnp.sqrt(d2 + 1e-8)

    col = jax.lax.broadcasted_iota(jnp.int32, (TILE, N), 1)
    row = jax.lax.broadcasted_iota(jnp.int32, (TILE, 1), 0) + i * TILE
    dm = jnp.where(col == row, 1e9, dist)

    vt = jnp.concatenate([xj[0], xj[1], xj[2], mrow_ref[0]], axis=0)
    hi = vt.astype(jnp.bfloat16)
    rem = vt - hi.astype(jnp.float32)
    mid = rem.astype(jnp.bfloat16)
    lo = (rem - mid.astype(jnp.float32)).astype(jnp.bfloat16)
    vt3 = jnp.concatenate([hi, mid, lo], axis=0)

    mi = mcol_ref[0]
    cen = cen_ref[0:1, :]
    be = be_ref[0:1, :]
    we_rbf = we_rbf_ref[...]
    we_dir = we_dir_ref[...]

    for k in range(K):
        m = jnp.min(dm, axis=1, keepdims=True)
        idx = jnp.min(jnp.where(dm == m, col, N), axis=1,
                      keepdims=True)
        onehot = col == idx
        dm = jnp.where(onehot, 1e9, dm)
        g = jax.lax.dot_general(
            onehot.astype(jnp.bfloat16), vt3, (((1,), (1,)), ((), ())),
            preferred_element_type=jnp.float32)
        dx = ((g[:, 0:1] + g[:, 4:5]) + g[:, 8:9]) - xi[0]
        dy = ((g[:, 1:2] + g[:, 5:6]) + g[:, 9:10]) - xi[1]
        dz = ((g[:, 2:3] + g[:, 6:7]) + g[:, 10:11]) - xi[2]
        mj = g[:, 3:4]
        inv = 1.0 / (m + 1e-8)
        z = (m - cen) / SIGMA
        rbf = jnp.exp(-(z * z))
        eh = jax.lax.dot_general(
            rbf, we_rbf, (((1,), (0,)), ((), ())),
            preferred_element_type=jnp.float32)
        eh = eh + ((dx * inv) * we_dir[0:1, :]
                   + (dy * inv) * we_dir[1:2, :]
                   + (dz * inv) * we_dir[2:3, :]) + be
        mij = mi * mj
        eh_ref[0, :, k * DIM_EDGES:(k + 1) * DIM_EDGES] = eh * mij
        ei_ref[0, :, k:k + 1] = idx
        mij_ref[0, :, k:k + 1] = mij

    xc3 = jnp.concatenate([xi[0], xi[1], xi[2]], axis=1)
    cen4 = jnp.concatenate([xc3, xc3, xc3, xc3], axis=1)
    centered = x12 - cen4
    lens = []
    for a in range(A):
        ca = centered[:, a * 3:a * 3 + 3]
        lens.append(jnp.sqrt(jnp.sum(ca * ca, axis=1, keepdims=True)))
    logl = jnp.log(jnp.concatenate(lens, axis=1) + 1e-6)
    nh = jax.lax.dot_general(
        centered, wn_c_ref[...], (((1,), (0,)), ((), ())),
        preferred_element_type=jnp.float32)
    nh = nh + jax.lax.dot_general(
        logl, wn_l_ref[...], (((1,), (0,)), ((), ())),
        preferred_element_type=jnp.float32)
    nh_ref[0] = (nh + bn_ref[0:1, :]) * mi


@functools.partial(jax.jit, static_argnames=())
def kernel(X, C, W_node, b_node, W_edge, b_edge):
    x12 = X.reshape(B, N, A * 3)
    xt = x12.transpose(0, 2, 1)
    mask_i = (C >= 0).astype(jnp.float32)
    mrow = mask_i.reshape(B, 1, N)
    mcol = mask_i.reshape(B, N, 1)
    centers = jnp.linspace(0.0, 20.0, NUM_RBF).astype(jnp.float32).reshape(1, NUM_RBF)

    grid = (B, N // TILE)
    node_h, edge_big, edge_idx, mask_ij = pl.pallas_call(
        _body,
        grid=grid,
        in_specs=[
            pl.BlockSpec((1, TILE, A * 3), lambda b, i: (b, i, 0)),
            pl.BlockSpec((1, A * 3, N), lambda b, i: (b, 0, 0)),
            pl.BlockSpec((1, 1, N), lambda b, i: (b, 0, 0)),
            pl.BlockSpec((1, TILE, 1), lambda b, i: (b, i, 0)),
            pl.BlockSpec((A * 3, DIM_NODES), lambda b, i: (0, 0)),
            pl.BlockSpec((A, DIM_NODES), lambda b, i: (0, 0)),
            pl.BlockSpec((NUM_RBF, DIM_EDGES), lambda b, i: (0, 0)),
            pl.BlockSpec((3, DIM_EDGES), lambda b, i: (0, 0)),
            pl.BlockSpec((1, DIM_NODES), lambda b, i: (0, 0)),
            pl.BlockSpec((1, DIM_EDGES), lambda b, i: (0, 0)),
            pl.BlockSpec((1, NUM_RBF), lambda b, i: (0, 0)),
        ],
        out_specs=[
            pl.BlockSpec((1, TILE, DIM_NODES), lambda b, i: (b, i, 0)),
            pl.BlockSpec((1, TILE, K * DIM_EDGES), lambda b, i: (b, i, 0)),
            pl.BlockSpec((1, TILE, K), lambda b, i: (b, i, 0)),
            pl.BlockSpec((1, TILE, K), lambda b, i: (b, i, 0)),
        ],
        out_shape=[
            jax.ShapeDtypeStruct((B, N, DIM_NODES), jnp.float32),
            jax.ShapeDtypeStruct((B, N, K * DIM_EDGES), jnp.float32),
            jax.ShapeDtypeStruct((B, N, K), jnp.int32),
            jax.ShapeDtypeStruct((B, N, K), jnp.float32),
        ],
    )(x12, xt, mrow, mcol,
      W_node[:A * 3], W_node[A * 3:],
      W_edge[:NUM_RBF], W_edge[NUM_RBF:],
      b_node.reshape(1, DIM_NODES), b_edge.reshape(1, DIM_EDGES), centers)

    edge_h = edge_big.reshape(B, N, K, DIM_EDGES)
    return node_h, edge_h, edge_idx, mask_i, mask_ij


# speedup vs baseline: 1.3716x; 1.1767x over previous
import functools

import jax
import jax.numpy as jnp
from jax.experimental import pallas as pl
from jax.experimental.pallas import tpu as pltpu

B, N, A = 4, 2048, 4
DIM_NODES, DIM_EDGES, K, NUM_RBF = 256, 128, 30, 32
TILE = 256
SIGMA = 20.0 / NUM_RBF


def _body(x12_ref, xt_ref, mrow_ref, mcol_ref, wn_c_ref, wn_l_ref,
          we_rbf_ref, we_dir_ref, bn_ref, be_ref, cen_ref,
          nh_ref, eh_ref, ei_ref, mij_ref):
    i = pl.program_id(1)
    x12 = x12_ref[0]
    xt = xt_ref[0]

    xi = [(((x12[:, c:c + 1] + x12[:, 3 + c:4 + c])
            + x12[:, 6 + c:7 + c]) + x12[:, 9 + c:10 + c]) * 0.25
          for c in range(3)]
    xj = [(((xt[c:c + 1, :] + xt[3 + c:4 + c, :])
            + xt[6 + c:7 + c, :]) + xt[9 + c:10 + c, :]) * 0.25
          for c in range(3)]

    d2 = ((xi[0] - xj[0]) ** 2 + (xi[2] - xj[2]) ** 2) + (xi[1] - xj[1]) ** 2
    dist = jnp.sqrt(d2 + 1e-8)

    col = jax.lax.broadcasted_iota(jnp.int32, (TILE, N), 1)
    row = jax.lax.broadcasted_iota(jnp.int32, (TILE, 1), 0) + i * TILE
    dm = jnp.where(col == row, 1e9, dist)

    vt = jnp.concatenate([xj[0], xj[1], xj[2], mrow_ref[0]], axis=0)
    hi = vt.astype(jnp.bfloat16)
    rem = vt - hi.astype(jnp.float32)
    mid = rem.astype(jnp.bfloat16)
    lo = (rem - mid.astype(jnp.float32)).astype(jnp.bfloat16)
    vt3 = jnp.concatenate([hi, mid, lo], axis=0)

    mi = mcol_ref[0]
    cen = cen_ref[0:1, :]
    be = be_ref[0:1, :]
    we_rbf = we_rbf_ref[...]
    we_dir = we_dir_ref[...]

    for k in range(K):
        m = jnp.min(dm, axis=1, keepdims=True)
        idx = jnp.min(jnp.where(dm == m, col, N), axis=1,
                      keepdims=True)
        onehot = col == idx
        dm = jnp.where(onehot, 1e9, dm)
        g = jax.lax.dot_general(
            onehot.astype(jnp.bfloat16), vt3, (((1,), (1,)), ((), ())),
            preferred_element_type=jnp.float32)
        dx = ((g[:, 0:1] + g[:, 4:5]) + g[:, 8:9]) - xi[0]
        dy = ((g[:, 1:2] + g[:, 5:6]) + g[:, 9:10]) - xi[1]
        dz = ((g[:, 2:3] + g[:, 6:7]) + g[:, 10:11]) - xi[2]
        mj = g[:, 3:4]
        inv = 1.0 / (m + 1e-8)
        z = (m - cen) / SIGMA
        rbf = jnp.exp(-(z * z))
        eh = jax.lax.dot_general(
            rbf, we_rbf, (((1,), (0,)), ((), ())),
            preferred_element_type=jnp.float32)
        eh = eh + ((dx * inv) * we_dir[0:1, :]
                   + (dy * inv) * we_dir[1:2, :]
                   + (dz * inv) * we_dir[2:3, :]) + be
        mij = mi * mj
        eh_ref[0, :, k * DIM_EDGES:(k + 1) * DIM_EDGES] = eh * mij
        ei_ref[0, :, k:k + 1] = idx
        mij_ref[0, :, k:k + 1] = mij

    xc3 = jnp.concatenate([xi[0], xi[1], xi[2]], axis=1)
    cen4 = jnp.concatenate([xc3, xc3, xc3, xc3], axis=1)
    centered = x12 - cen4
    lens = []
    for a in range(A):
        ca = centered[:, a * 3:a * 3 + 3]
        lens.append(jnp.sqrt(jnp.sum(ca * ca, axis=1, keepdims=True)))
    logl = jnp.log(jnp.concatenate(lens, axis=1) + 1e-6)
    nh = jax.lax.dot_general(
        centered, wn_c_ref[...], (((1,), (0,)), ((), ())),
        preferred_element_type=jnp.float32)
    nh = nh + jax.lax.dot_general(
        logl, wn_l_ref[...], (((1,), (0,)), ((), ())),
        preferred_element_type=jnp.float32)
    nh_ref[0] = (nh + bn_ref[0:1, :]) * mi


@functools.partial(jax.jit, static_argnames=())
def kernel(X, C, W_node, b_node, W_edge, b_edge):
    x12 = X.reshape(B, N, A * 3)
    xt = x12.transpose(0, 2, 1)
    mask_i = (C >= 0).astype(jnp.float32)
    mrow = mask_i.reshape(B, 1, N)
    mcol = mask_i.reshape(B, N, 1)
    centers = jnp.linspace(0.0, 20.0, NUM_RBF).astype(jnp.float32).reshape(1, NUM_RBF)

    grid = (B, N // TILE)
    node_h, edge_big, edge_idx, mask_ij = pl.pallas_call(
        _body,
        grid=grid,
        in_specs=[
            pl.BlockSpec((1, TILE, A * 3), lambda b, i: (b, i, 0)),
            pl.BlockSpec((1, A * 3, N), lambda b, i: (b, 0, 0)),
            pl.BlockSpec((1, 1, N), lambda b, i: (b, 0, 0)),
            pl.BlockSpec((1, TILE, 1), lambda b, i: (b, i, 0)),
            pl.BlockSpec((A * 3, DIM_NODES), lambda b, i: (0, 0)),
            pl.BlockSpec((A, DIM_NODES), lambda b, i: (0, 0)),
            pl.BlockSpec((NUM_RBF, DIM_EDGES), lambda b, i: (0, 0)),
            pl.BlockSpec((3, DIM_EDGES), lambda b, i: (0, 0)),
            pl.BlockSpec((1, DIM_NODES), lambda b, i: (0, 0)),
            pl.BlockSpec((1, DIM_EDGES), lambda b, i: (0, 0)),
            pl.BlockSpec((1, NUM_RBF), lambda b, i: (0, 0)),
        ],
        out_specs=[
            pl.BlockSpec((1, TILE, DIM_NODES), lambda b, i: (b, i, 0)),
            pl.BlockSpec((1, TILE, K * DIM_EDGES), lambda b, i: (b, i, 0)),
            pl.BlockSpec((1, TILE, K), lambda b, i: (b, i, 0)),
            pl.BlockSpec((1, TILE, K), lambda b, i: (b, i, 0)),
        ],
        out_shape=[
            jax.ShapeDtypeStruct((B, N, DIM_NODES), jnp.float32),
            jax.ShapeDtypeStruct((B, N, K * DIM_EDGES), jnp.float32),
            jax.ShapeDtypeStruct((B, N, K), jnp.int32),
            jax.ShapeDtypeStruct((B, N, K), jnp.float32),
        ],
    )(x12, xt, mrow, mcol,
      W_node[:A * 3], W_node[A * 3:],
      W_edge[:NUM_RBF], W_edge[NUM_RBF:],
      b_node.reshape(1, DIM_NODES), b_edge.reshape(1, DIM_EDGES), centers)

    edge_h = edge_big.reshape(B, N, K, DIM_EDGES)
    return node_h, edge_h, edge_idx, mask_i, mask_ij
